# baseline (speedup 1.0000x reference)
import numpy as np

import jax
import jax.numpy as jnp
from jax.experimental import pallas as pl
from jax.experimental.pallas import tpu as pltpu

_C1 = 32
_C2 = 64
_NCLS = 10
_BBL = 1024
_M1 = 13 * _C1


def _fused_kernel(x_ref, a1_ref, b1r_ref, a2_ref, b2r_ref,
                  wf1_ref, bf1_ref, wf2_ref, bf2_ref, out_ref,
                  xt_ref, p1_ref, f_ref):
    a1 = a1_ref[...]
    b1r = b1r_ref[...]

    xt_ref[...] = jnp.transpose(x_ref[...].astype(jnp.bfloat16))

    for hp in range(13):
        xa = xt_ref[56 * hp: 56 * hp + 84, :]
        xb = xt_ref[56 * hp + 28: 56 * hp + 112, :]
        r0 = jnp.dot(a1, xa, preferred_element_type=jnp.float32)
        r1 = jnp.dot(a1, xb, preferred_element_type=jnp.float32)
        m = jnp.maximum(jnp.maximum(r0[:_M1], r0[_M1:]),
                        jnp.maximum(r1[:_M1], r1[_M1:]))
        pooled = jnp.maximum(m + b1r[:_M1], 0.0)
        p1_ref[_M1 * hp: _M1 * (hp + 1), :] = pooled.astype(jnp.bfloat16)

    a2 = a2_ref[...]
    b2r = b2r_ref[...]
    for ph in range(5):
        s0 = 416 * (2 * ph)
        u0 = jnp.dot(a2, p1_ref[s0: s0 + 1248, :],
                     preferred_element_type=jnp.float32)
        u1 = jnp.dot(a2, p1_ref[s0 + 416: s0 + 1664, :],
                     preferred_element_type=jnp.float32)
        m = jnp.maximum(u0, u1)
        b2s = b2r[0:_C2]
        for pw in range(5):
            mm = jnp.maximum(m[128 * pw: 128 * pw + 64],
                             m[128 * pw + 64: 128 * pw + 128])
            pooled = jnp.maximum(mm + b2s, 0.0)
            base = (ph * 5 + pw) * _C2
            f_ref[base: base + _C2, :] = pooled.astype(jnp.bfloat16)

    h = jnp.dot(wf1_ref[...], f_ref[...], preferred_element_type=jnp.float32)
    h = jnp.maximum(h + bf1_ref[...], 0.0)
    logits = jnp.dot(wf2_ref[...], h.astype(jnp.bfloat16),
                     preferred_element_type=jnp.float32)
    logits = logits + bf2_ref[...]
    z = logits - jnp.max(logits, axis=0, keepdims=True)
    e = jnp.exp(z)
    inv = pl.reciprocal(jnp.sum(e, axis=0, keepdims=True), approx=True)
    out_ref[...] = e * inv


_OH = np.zeros((2, 13, 9, 84), np.float32)
for _p in range(2):
    for _w in range(13):
        for _k in range(9):
            _OH[_p, _w, _k, 28 * (_k // 3) + 2 * _w + _p + _k % 3] = 1.0


def _build_toeplitz(w1m):
    a = jnp.einsum('pwkj,kc->pwcj', jnp.asarray(_OH), w1m)
    return a.reshape(2 * _M1, 84).astype(jnp.bfloat16)


_OH2 = np.zeros((10, 3, 13), np.float32)
for _w2 in range(10):
    for _kj in range(3):
        _OH2[_w2, _kj, _w2 + _kj] = 1.0


def _build_conv2_toeplitz(w2m):
    w4 = w2m.reshape(3, 3, _C1, _C2)
    a2 = jnp.einsum('wjv,ijcd->wdivc', jnp.asarray(_OH2), w4)
    return a2.reshape(640, 1248).astype(jnp.bfloat16)


def kernel(w1, b1, w2, b2, wf1, bf1, wf2, bf2, x):
    B = x.shape[0]
    x2d = x.reshape(B, 784)

    a1 = _build_toeplitz(w1)
    b1r = jnp.tile(b1.reshape(_C1), 26).reshape(2 * _M1, 1)
    a2 = _build_conv2_toeplitz(w2)
    b2r = jnp.tile(b2.reshape(_C2), 10).reshape(640, 1)
    wf1t = jnp.transpose(wf1).astype(jnp.bfloat16)
    bf1c = bf1.reshape(128, 1)
    wf2t = jnp.transpose(wf2).astype(jnp.bfloat16)
    bf2c = bf2.reshape(128, 1)

    grid = (B // _BBL,)
    flops = B * (2 * 26 * 26 * 9 * _C1 + 2 * 100 * 9 * _C1 * _C2
                 + 2 * 1600 * 128 + 2 * 128 * 128)
    probs_t = pl.pallas_call(
        _fused_kernel,
        out_shape=jax.ShapeDtypeStruct((128, B), jnp.float32),
        grid_spec=pltpu.PrefetchScalarGridSpec(
            num_scalar_prefetch=0,
            grid=grid,
            in_specs=[
                pl.BlockSpec((_BBL, 784), lambda i: (i, 0)),
                pl.BlockSpec((2 * _M1, 84), lambda i: (0, 0)),
                pl.BlockSpec((2 * _M1, 1), lambda i: (0, 0)),
                pl.BlockSpec((640, 1248), lambda i: (0, 0)),
                pl.BlockSpec((640, 1), lambda i: (0, 0)),
                pl.BlockSpec((128, 1600), lambda i: (0, 0)),
                pl.BlockSpec((128, 1), lambda i: (0, 0)),
                pl.BlockSpec((128, 128), lambda i: (0, 0)),
                pl.BlockSpec((128, 1), lambda i: (0, 0)),
            ],
            out_specs=pl.BlockSpec((128, _BBL), lambda i: (0, i)),
            scratch_shapes=[
                pltpu.VMEM((784, _BBL), jnp.bfloat16),
                pltpu.VMEM((13 * _M1, _BBL), jnp.bfloat16),
                pltpu.VMEM((1600, _BBL), jnp.bfloat16),
            ],
        ),
        compiler_params=pltpu.CompilerParams(
            dimension_semantics=("parallel",),
            vmem_limit_bytes=64 * 1024 * 1024,
        ),
        cost_estimate=pl.CostEstimate(
            flops=flops, transcendentals=B * 128,
            bytes_accessed=2 * B * 784 + 4 * B * 128),
    )(x2d, a1, b1r, a2, b2r, wf1t, bf1c, wf2t, bf2c)

    return jnp.transpose(probs_t[:_NCLS, :])


# speedup vs baseline: 1.2125x; 1.2125x over previous
import numpy as np

import jax
import jax.numpy as jnp
from jax.experimental import pallas as pl
from jax.experimental.pallas import tpu as pltpu

_C1 = 32
_C2 = 64
_NCLS = 10
_BBL = 1024
_M1 = 13 * _C1


def _fused_kernel(x_ref, a1_ref, b1r_ref, a2a_ref, a2b_ref, b2r_ref,
                  wf1_ref, bf1_ref, wf2_ref, bf2_ref, out_ref,
                  xt_ref, p1_ref, f_ref):
    a1 = a1_ref[...]
    b1r = b1r_ref[...]

    xt_ref[...] = jnp.transpose(x_ref[...].astype(jnp.bfloat16))

    for hp in range(13):
        xa = xt_ref[56 * hp: 56 * hp + 84, :]
        xb = xt_ref[56 * hp + 28: 56 * hp + 112, :]
        r0 = jnp.dot(a1, xa, preferred_element_type=jnp.float32)
        r1 = jnp.dot(a1, xb, preferred_element_type=jnp.float32)
        m = jnp.maximum(jnp.maximum(r0[:_M1], r0[_M1:]),
                        jnp.maximum(r1[:_M1], r1[_M1:]))
        pooled = jnp.maximum(m + b1r[:_M1], 0.0)
        p1_ref[_M1 * hp: _M1 * (hp + 1), :] = pooled.astype(jnp.bfloat16)

    a2a = a2a_ref[...]
    a2b = a2b_ref[...]
    b2s = b2r_ref[...]

    def conv2_row(h2):
        ua = jnp.dot(a2a[:, 0:256], p1_ref[416 * h2: 416 * h2 + 256, :],
                     preferred_element_type=jnp.float32)
        ub = jnp.dot(a2b[:, 0:192], p1_ref[416 * h2 + 192: 416 * h2 + 384, :],
                     preferred_element_type=jnp.float32)
        for ki in (1, 2):
            s = 416 * (h2 + ki)
            ua = ua + jnp.dot(a2a[:, 256 * ki: 256 * ki + 256],
                              p1_ref[s: s + 256, :],
                              preferred_element_type=jnp.float32)
            ub = ub + jnp.dot(a2b[:, 192 * ki: 192 * ki + 192],
                              p1_ref[s + 192: s + 384, :],
                              preferred_element_type=jnp.float32)
        return ua, ub

    for ph in range(5):
        ua0, ub0 = conv2_row(2 * ph)
        ua1, ub1 = conv2_row(2 * ph + 1)
        ma = jnp.maximum(ua0, ua1)
        mb = jnp.maximum(ub0, ub1)
        for pw in range(3):
            mm = jnp.maximum(ma[128 * pw: 128 * pw + 64],
                             ma[128 * pw + 64: 128 * pw + 128])
            pooled = jnp.maximum(mm + b2s, 0.0)
            base = (ph * 5 + pw) * _C2
            f_ref[base: base + _C2, :] = pooled.astype(jnp.bfloat16)
        for pw in (3, 4):
            o = 128 * (pw - 3)
            mm = jnp.maximum(mb[o: o + 64], mb[o + 64: o + 128])
            pooled = jnp.maximum(mm + b2s, 0.0)
            base = (ph * 5 + pw) * _C2
            f_ref[base: base + _C2, :] = pooled.astype(jnp.bfloat16)

    h = jnp.dot(wf1_ref[...], f_ref[...], preferred_element_type=jnp.float32)
    h = jnp.maximum(h + bf1_ref[...], 0.0)
    logits = jnp.dot(wf2_ref[...], h.astype(jnp.bfloat16),
                     preferred_element_type=jnp.float32)
    logits = logits + bf2_ref[...]
    z = logits - jnp.max(logits, axis=0, keepdims=True)
    e = jnp.exp(z)
    inv = pl.reciprocal(jnp.sum(e, axis=0, keepdims=True), approx=True)
    out_ref[...] = e * inv


_OH = np.zeros((2, 13, 9, 84), np.float32)
for _p in range(2):
    for _w in range(13):
        for _k in range(9):
            _OH[_p, _w, _k, 28 * (_k // 3) + 2 * _w + _p + _k % 3] = 1.0


def _build_toeplitz(w1m):
    a = jnp.einsum('pwkj,kc->pwcj', jnp.asarray(_OH), w1m)
    return a.reshape(2 * _M1, 84).astype(jnp.bfloat16)


_OH2A = np.zeros((6, 3, 8), np.float32)
for _w2 in range(6):
    for _kj in range(3):
        _OH2A[_w2, _kj, _w2 + _kj] = 1.0
_OH2B = np.zeros((4, 3, 6), np.float32)
for _w2 in range(4):
    for _kj in range(3):
        _OH2B[_w2, _kj, _w2 + _kj] = 1.0


def _build_conv2_toeplitz(w2m):
    w4 = w2m.reshape(3, 3, _C1, _C2)
    a2a = jnp.einsum('wjv,ijcd->wdivc', jnp.asarray(_OH2A), w4)
    a2b = jnp.einsum('wjv,ijcd->wdivc', jnp.asarray(_OH2B), w4)
    return (a2a.reshape(384, 768).astype(jnp.bfloat16),
            a2b.reshape(256, 576).astype(jnp.bfloat16))


def kernel(w1, b1, w2, b2, wf1, bf1, wf2, bf2, x):
    B = x.shape[0]
    x2d = x.reshape(B, 784)

    a1 = _build_toeplitz(w1)
    b1r = jnp.tile(b1.reshape(_C1), 26).reshape(2 * _M1, 1)
    a2a, a2b = _build_conv2_toeplitz(w2)
    b2r = b2.reshape(_C2, 1)
    wf1t = jnp.transpose(wf1).astype(jnp.bfloat16)
    bf1c = bf1.reshape(128, 1)
    wf2t = jnp.transpose(wf2).astype(jnp.bfloat16)
    bf2c = bf2.reshape(128, 1)

    grid = (B // _BBL,)
    flops = B * (2 * 26 * 26 * 9 * _C1 + 2 * 100 * 9 * _C1 * _C2
                 + 2 * 1600 * 128 + 2 * 128 * 128)
    probs_t = pl.pallas_call(
        _fused_kernel,
        out_shape=jax.ShapeDtypeStruct((128, B), jnp.float32),
        grid_spec=pltpu.PrefetchScalarGridSpec(
            num_scalar_prefetch=0,
            grid=grid,
            in_specs=[
                pl.BlockSpec((_BBL, 784), lambda i: (i, 0)),
                pl.BlockSpec((2 * _M1, 84), lambda i: (0, 0)),
                pl.BlockSpec((2 * _M1, 1), lambda i: (0, 0)),
                pl.BlockSpec((384, 768), lambda i: (0, 0)),
                pl.BlockSpec((256, 576), lambda i: (0, 0)),
                pl.BlockSpec((_C2, 1), lambda i: (0, 0)),
                pl.BlockSpec((128, 1600), lambda i: (0, 0)),
                pl.BlockSpec((128, 1), lambda i: (0, 0)),
                pl.BlockSpec((128, 128), lambda i: (0, 0)),
                pl.BlockSpec((128, 1), lambda i: (0, 0)),
            ],
            out_specs=pl.BlockSpec((128, _BBL), lambda i: (0, i)),
            scratch_shapes=[
                pltpu.VMEM((784, _BBL), jnp.bfloat16),
                pltpu.VMEM((13 * _M1, _BBL), jnp.bfloat16),
                pltpu.VMEM((1600, _BBL), jnp.bfloat16),
            ],
        ),
        compiler_params=pltpu.CompilerParams(
            dimension_semantics=("parallel",),
            vmem_limit_bytes=64 * 1024 * 1024,
        ),
        cost_estimate=pl.CostEstimate(
            flops=flops, transcendentals=B * 128,
            bytes_accessed=2 * B * 784 + 4 * B * 128),
    )(x2d, a1, b1r, a2a, a2b, b2r, wf1t, bf1c, wf2t, bf2c)

    return jnp.transpose(probs_t[:_NCLS, :])


# speedup vs baseline: 1.3329x; 1.0993x over previous
import numpy as np

import jax
import jax.numpy as jnp
from jax.experimental import pallas as pl
from jax.experimental.pallas import tpu as pltpu

_C1 = 32
_C2 = 64
_NCLS = 10
_BBL = 1024
_M1 = 13 * _C1


def _fused_kernel(xt_ref, a1_ref, b1r_ref, a2a_ref, a2b_ref, b2r_ref,
                  wf1_ref, bf1_ref, wf2_ref, bf2_ref, out_ref,
                  p1_ref, f_ref):
    a1 = a1_ref[...]
    b1r = b1r_ref[...]

    for hp in range(13):
        xa = xt_ref[56 * hp: 56 * hp + 84, :]
        xb = xt_ref[56 * hp + 28: 56 * hp + 112, :]
        r0 = jnp.dot(a1, xa, preferred_element_type=jnp.float32)
        r1 = jnp.dot(a1, xb, preferred_element_type=jnp.float32)
        m = jnp.maximum(jnp.maximum(r0[:_M1], r0[_M1:]),
                        jnp.maximum(r1[:_M1], r1[_M1:]))
        pooled = jnp.maximum(m + b1r[:_M1], 0.0)
        p1_ref[_M1 * hp: _M1 * (hp + 1), :] = pooled.astype(jnp.bfloat16)

    a2a = a2a_ref[...]
    a2b = a2b_ref[...]
    b2s = b2r_ref[...]

    def conv2_row(h2):
        ua = jnp.dot(a2a[:, 0:256], p1_ref[416 * h2: 416 * h2 + 256, :],
                     preferred_element_type=jnp.float32)
        ub = jnp.dot(a2b[:, 0:192], p1_ref[416 * h2 + 192: 416 * h2 + 384, :],
                     preferred_element_type=jnp.float32)
        for ki in (1, 2):
            s = 416 * (h2 + ki)
            ua = ua + jnp.dot(a2a[:, 256 * ki: 256 * ki + 256],
                              p1_ref[s: s + 256, :],
                              preferred_element_type=jnp.float32)
            ub = ub + jnp.dot(a2b[:, 192 * ki: 192 * ki + 192],
                              p1_ref[s + 192: s + 384, :],
                              preferred_element_type=jnp.float32)
        return ua, ub

    for ph in range(5):
        ua0, ub0 = conv2_row(2 * ph)
        ua1, ub1 = conv2_row(2 * ph + 1)
        ma = jnp.maximum(ua0, ua1)
        mb = jnp.maximum(ub0, ub1)
        for pw in range(3):
            mm = jnp.maximum(ma[128 * pw: 128 * pw + 64],
                             ma[128 * pw + 64: 128 * pw + 128])
            pooled = jnp.maximum(mm + b2s, 0.0)
            base = (ph * 5 + pw) * _C2
            f_ref[base: base + _C2, :] = pooled.astype(jnp.bfloat16)
        for pw in (3, 4):
            o = 128 * (pw - 3)
            mm = jnp.maximum(mb[o: o + 64], mb[o + 64: o + 128])
            pooled = jnp.maximum(mm + b2s, 0.0)
            base = (ph * 5 + pw) * _C2
            f_ref[base: base + _C2, :] = pooled.astype(jnp.bfloat16)

    h = jnp.dot(wf1_ref[...], f_ref[...], preferred_element_type=jnp.float32)
    h = jnp.maximum(h + bf1_ref[...], 0.0)
    logits = jnp.dot(wf2_ref[...], h.astype(jnp.bfloat16),
                     preferred_element_type=jnp.float32)
    logits = logits + bf2_ref[...]
    z = logits - jnp.max(logits, axis=0, keepdims=True)
    e = jnp.exp(z)
    inv = pl.reciprocal(jnp.sum(e, axis=0, keepdims=True), approx=True)
    out_ref[...] = e * inv


_OH = np.zeros((2, 13, 9, 84), np.float32)
for _p in range(2):
    for _w in range(13):
        for _k in range(9):
            _OH[_p, _w, _k, 28 * (_k // 3) + 2 * _w + _p + _k % 3] = 1.0


def _build_toeplitz(w1m):
    a = jnp.einsum('pwkj,kc->pwcj', jnp.asarray(_OH), w1m)
    return a.reshape(2 * _M1, 84).astype(jnp.bfloat16)


_OH2A = np.zeros((6, 3, 8), np.float32)
for _w2 in range(6):
    for _kj in range(3):
        _OH2A[_w2, _kj, _w2 + _kj] = 1.0
_OH2B = np.zeros((4, 3, 6), np.float32)
for _w2 in range(4):
    for _kj in range(3):
        _OH2B[_w2, _kj, _w2 + _kj] = 1.0


def _build_conv2_toeplitz(w2m):
    w4 = w2m.reshape(3, 3, _C1, _C2)
    a2a = jnp.einsum('wjv,ijcd->wdivc', jnp.asarray(_OH2A), w4)
    a2b = jnp.einsum('wjv,ijcd->wdivc', jnp.asarray(_OH2B), w4)
    return (a2a.reshape(384, 768).astype(jnp.bfloat16),
            a2b.reshape(256, 576).astype(jnp.bfloat16))


def kernel(w1, b1, w2, b2, wf1, bf1, wf2, bf2, x):
    B = x.shape[0]
    xt = jnp.transpose(x.astype(jnp.bfloat16).reshape(B, 784))

    a1 = _build_toeplitz(w1)
    b1r = jnp.tile(b1.reshape(_C1), 26).reshape(2 * _M1, 1)
    a2a, a2b = _build_conv2_toeplitz(w2)
    b2r = b2.reshape(_C2, 1)
    wf1t = jnp.transpose(wf1).astype(jnp.bfloat16)
    bf1c = bf1.reshape(128, 1)
    wf2t = jnp.transpose(wf2).astype(jnp.bfloat16)
    bf2c = bf2.reshape(128, 1)

    grid = (B // _BBL,)
    flops = B * (2 * 26 * 26 * 9 * _C1 + 2 * 100 * 9 * _C1 * _C2
                 + 2 * 1600 * 128 + 2 * 128 * 128)
    probs_t = pl.pallas_call(
        _fused_kernel,
        out_shape=jax.ShapeDtypeStruct((128, B), jnp.float32),
        grid_spec=pltpu.PrefetchScalarGridSpec(
            num_scalar_prefetch=0,
            grid=grid,
            in_specs=[
                pl.BlockSpec((784, _BBL), lambda i: (0, i)),
                pl.BlockSpec((2 * _M1, 84), lambda i: (0, 0)),
                pl.BlockSpec((2 * _M1, 1), lambda i: (0, 0)),
                pl.BlockSpec((384, 768), lambda i: (0, 0)),
                pl.BlockSpec((256, 576), lambda i: (0, 0)),
                pl.BlockSpec((_C2, 1), lambda i: (0, 0)),
                pl.BlockSpec((128, 1600), lambda i: (0, 0)),
                pl.BlockSpec((128, 1), lambda i: (0, 0)),
                pl.BlockSpec((128, 128), lambda i: (0, 0)),
                pl.BlockSpec((128, 1), lambda i: (0, 0)),
            ],
            out_specs=pl.BlockSpec((128, _BBL), lambda i: (0, i)),
            scratch_shapes=[
                pltpu.VMEM((13 * _M1, _BBL), jnp.bfloat16),
                pltpu.VMEM((1600, _BBL), jnp.bfloat16),
            ],
        ),
        compiler_params=pltpu.CompilerParams(
            dimension_semantics=("parallel",),
            vmem_limit_bytes=64 * 1024 * 1024,
        ),
        cost_estimate=pl.CostEstimate(
            flops=flops, transcendentals=B * 128,
            bytes_accessed=2 * B * 784 + 4 * B * 128),
    )(xt, a1, b1r, a2a, a2b, b2r, wf1t, bf1c, wf2t, bf2c)

    return jnp.transpose(probs_t[:_NCLS, :])
